# baseline (speedup 1.0000x reference)
import jax
import jax.numpy as jnp
from jax import lax
from jax.experimental import pallas as pl
from jax.experimental.pallas import tpu as pltpu
from jax.experimental.pallas import tpu_sc as plsc

N = 10000
E = 320000
NC, NS = 2, 16
NW = NC * NS
CW = 512
NBUF = 4
FAST = 0
KFB, KSB = 28, 12
NBLK = NS * (KFB + KSB)
EP = NBLK * CW
N_TBL = N + 16
ROWS_PER_TILE = 640
N_ACC = NS * ROWS_PER_TILE
D = 8


def _sc_mesh():
    return plsc.VectorSubcoreMesh(core_axis_name="c", subcore_axis_name="s")


_SC_PARAMS = pltpu.CompilerParams(use_tc_tiling_on_sc=False)


def _chunk_assignment(c, s):
    is_fast = c == FAST
    base = lax.select(is_fast, s * KFB, NS * KFB + s * KSB)
    kcb = lax.select(is_fast, KFB, KSB)
    return is_fast, base, kcb


def _load_idx(ei_hbm, d, is_fast, base, dst):
    @pl.when(is_fast)
    def _():
        pltpu.sync_copy(ei_hbm.at[d].at[pl.ds(base, KFB)], dst.at[pl.ds(0, KFB)])

    @pl.when(jnp.logical_not(is_fast))
    def _():
        pltpu.sync_copy(ei_hbm.at[d].at[pl.ds(base, KSB)], dst.at[pl.ds(0, KSB)])


def _deg_body(ei_hbm, ones_hbm, zeros_hbm, out_hbm, didx, ones_v, acc, sem):
    c = lax.axis_index("c")
    s = lax.axis_index("s")
    is_fast, base, kcb = _chunk_assignment(c, s)
    pltpu.sync_copy(zeros_hbm, acc.at[pl.ds(s * ROWS_PER_TILE, ROWS_PER_TILE)])
    pltpu.sync_copy(ones_hbm, ones_v)
    _load_idx(ei_hbm, 1, is_fast, base, didx)
    plsc.subcore_barrier()

    def body(i, _):
        for j in range(NBUF):
            pltpu.async_copy(ones_v, acc.at[didx.at[i * NBUF + j]], sem,
                             add=True)
        for j in range(NBUF):
            pltpu.make_async_copy(ones_v, acc.at[didx.at[i * NBUF + j]],
                                  sem).wait()
        return _

    lax.fori_loop(0, kcb // NBUF, body, None)
    plsc.subcore_barrier()
    sl = pl.ds(s * ROWS_PER_TILE, ROWS_PER_TILE)
    pltpu.sync_copy(acc.at[sl], out_hbm.at[c].at[sl])


def _prop_body(table_hbm, ei_hbm, zeros_hbm, out_hbm,
               sidx, didx, rows, acc, gsem, ssem):
    c = lax.axis_index("c")
    s = lax.axis_index("s")
    is_fast, base, kcb = _chunk_assignment(c, s)
    pltpu.sync_copy(zeros_hbm, acc.at[pl.ds(s * ROWS_PER_TILE, ROWS_PER_TILE)])
    _load_idx(ei_hbm, 0, is_fast, base, sidx)
    _load_idx(ei_hbm, 1, is_fast, base, didx)
    plsc.subcore_barrier()

    for j in range(NBUF):
        pltpu.async_copy(table_hbm.at[sidx.at[j]], rows.at[j], gsem.at[j])

    def body(i, _):
        for j in range(NBUF):
            k = i * NBUF + j
            pltpu.make_async_copy(table_hbm.at[sidx.at[k]], rows.at[j],
                                  gsem.at[j]).wait()
            pltpu.async_copy(rows.at[j], acc.at[didx.at[k]], ssem.at[j],
                             add=True)
            nk = k + NBUF

            @pl.when(nk < kcb)
            def _prefetch():
                pltpu.make_async_copy(rows.at[j], acc.at[didx.at[k]],
                                      ssem.at[j]).wait()
                pltpu.async_copy(table_hbm.at[sidx.at[nk]], rows.at[j],
                                 gsem.at[j])
        return _

    lax.fori_loop(0, kcb // NBUF, body, None)
    for j in range(NBUF):
        pltpu.make_async_copy(rows.at[j], acc.at[didx.at[kcb - NBUF + j]],
                              ssem.at[j]).wait()
    plsc.subcore_barrier()
    sl = pl.ds(s * ROWS_PER_TILE, ROWS_PER_TILE)
    pltpu.sync_copy(acc.at[sl], out_hbm.at[c].at[sl])


def _sc_degree(ei_r, ones128, zeros640):
    return pl.kernel(
        _deg_body,
        out_type=jax.ShapeDtypeStruct((NC, N_ACC, D), jnp.float32),
        mesh=_sc_mesh(),
        compiler_params=_SC_PARAMS,
        scratch_types=[
            pltpu.VMEM((KFB, CW), jnp.int32),
            pltpu.VMEM((CW, D), jnp.float32),
            pltpu.VMEM_SHARED((N_ACC, D), jnp.float32),
            pltpu.SemaphoreType.DMA,
        ],
    )(ei_r, ones128, zeros640)


def _sc_propagate(table, ei_r, zeros640):
    return pl.kernel(
        _prop_body,
        out_type=jax.ShapeDtypeStruct((NC, N_ACC, D), jnp.float32),
        mesh=_sc_mesh(),
        compiler_params=_SC_PARAMS,
        scratch_types=[
            pltpu.VMEM((KFB, CW), jnp.int32),
            pltpu.VMEM((KFB, CW), jnp.int32),
            pltpu.VMEM((NBUF, CW, D), jnp.float32),
            pltpu.VMEM_SHARED((N_ACC, D), jnp.float32),
            pltpu.SemaphoreType.DMA((NBUF,)),
            pltpu.SemaphoreType.DMA((NBUF,)),
        ],
    )(table, ei_r, zeros640)


def _tca_body(x_ref, w1_ref, h_ref):
    h_ref[...] = jnp.dot(x_ref[...], w1_ref[...],
                         preferred_element_type=jnp.float32)


def _tcb_body(degp_ref, h_ref, hp1_ref, dis_ref):
    deg = degp_ref[0, : N, 0:1] + degp_ref[1, : N, 0:1] + 1.0
    dis = lax.rsqrt(deg)
    hp1_ref[pl.ds(0, N), :] = dis * h_ref[...]
    hp1_ref[pl.ds(N, N_TBL - N), :] = jnp.zeros((N_TBL - N, D), jnp.float32)
    dis_ref[...] = dis


def _tc2_body(p1_ref, hp1_ref, dis_ref, b1_ref, q_ref):
    s1 = p1_ref[0, : N, :] + p1_ref[1, : N, :]
    dis = dis_ref[...]
    pre = dis * (s1 + hp1_ref[pl.ds(0, N), :]) + b1_ref[...]
    q_ref[pl.ds(0, N), :] = dis * jnp.maximum(pre, 0.0)
    q_ref[pl.ds(N, N_TBL - N), :] = jnp.zeros((N_TBL - N, D), jnp.float32)


def _tc3_body(p2_ref, q_ref, dis_ref, w2_ref, b2_ref, wc_ref, bc_ref, out_ref):
    s2 = p2_ref[0, : N, :] + p2_ref[1, : N, :]
    t = s2 + q_ref[pl.ds(0, N), :]
    conv2 = dis_ref[...] * jnp.dot(
        t, w2_ref[...], preferred_element_type=jnp.float32) + b2_ref[...]
    emb = jnp.maximum(conv2, 0.0)
    z = jnp.dot(emb, wc_ref[...], preferred_element_type=jnp.float32) + bc_ref[...]
    out_ref[...] = jax.nn.sigmoid(z)


def _tc_call(body, out_shapes, *args):
    return pl.pallas_call(body, out_shape=out_shapes)(*args)


@jax.jit
def kernel(x, edge_index, W1, b1, W2, b2, Wc, bc):
    ei_r = jnp.pad(edge_index.astype(jnp.int32), ((0, 0), (0, EP - E)),
                   constant_values=N).reshape(2, NBLK, CW)
    zeros640 = jnp.zeros((ROWS_PER_TILE, D), jnp.float32)
    ones128 = jnp.ones((CW, D), jnp.float32)

    h = _tc_call(_tca_body, jax.ShapeDtypeStruct((N, D), jnp.float32), x, W1)
    degp = _sc_degree(ei_r, ones128, zeros640)
    hp1, dis = _tc_call(
        _tcb_body,
        [jax.ShapeDtypeStruct((N_TBL, D), jnp.float32),
         jax.ShapeDtypeStruct((N, 1), jnp.float32)],
        degp, h)
    p1 = _sc_propagate(hp1, ei_r, zeros640)
    q = _tc_call(
        _tc2_body,
        jax.ShapeDtypeStruct((N_TBL, D), jnp.float32),
        p1, hp1, dis, b1.reshape(1, D))
    p2 = _sc_propagate(q, ei_r, zeros640)
    out = _tc_call(
        _tc3_body,
        jax.ShapeDtypeStruct((N, 1), jnp.float32),
        p2, q, dis, W2, b2.reshape(1, 2), Wc, bc.reshape(1, 1))
    return out


# speedup vs baseline: 1.0232x; 1.0232x over previous
import jax
import jax.numpy as jnp
from jax import lax
from jax.experimental import pallas as pl
from jax.experimental.pallas import tpu as pltpu
from jax.experimental.pallas import tpu_sc as plsc

N = 10000
E = 320000
NC, NS = 2, 16
NW = NC * NS
CW = 128
NBUF = 8
FAST = 0
KFB, KSB = 112, 48
NBLK = NS * (KFB + KSB)
EP = NBLK * CW
N_TBL = N + 16
ROWS_PER_TILE = 640
N_ACC = NS * ROWS_PER_TILE
D = 8


def _sc_mesh():
    return plsc.VectorSubcoreMesh(core_axis_name="c", subcore_axis_name="s")


_SC_PARAMS = pltpu.CompilerParams(use_tc_tiling_on_sc=False)


def _chunk_assignment(c, s):
    is_fast = c == FAST
    base = lax.select(is_fast, s * KFB, NS * KFB + s * KSB)
    kcb = lax.select(is_fast, KFB, KSB)
    return is_fast, base, kcb


def _load_idx(ei_hbm, d, is_fast, base, dst):
    @pl.when(is_fast)
    def _():
        pltpu.sync_copy(ei_hbm.at[d].at[pl.ds(base, KFB)], dst.at[pl.ds(0, KFB)])

    @pl.when(jnp.logical_not(is_fast))
    def _():
        pltpu.sync_copy(ei_hbm.at[d].at[pl.ds(base, KSB)], dst.at[pl.ds(0, KSB)])


def _deg_body(ei_hbm, ones_hbm, zeros_hbm, out_hbm, didx, ones_v, acc, sem):
    c = lax.axis_index("c")
    s = lax.axis_index("s")
    is_fast, base, kcb = _chunk_assignment(c, s)
    pltpu.sync_copy(zeros_hbm, acc.at[pl.ds(s * ROWS_PER_TILE, ROWS_PER_TILE)])
    pltpu.sync_copy(ones_hbm, ones_v)
    _load_idx(ei_hbm, 1, is_fast, base, didx)
    plsc.subcore_barrier()

    def body(i, _):
        for j in range(NBUF):
            pltpu.async_copy(ones_v, acc.at[didx.at[i * NBUF + j]], sem,
                             add=True)
        for j in range(NBUF):
            pltpu.make_async_copy(ones_v, acc.at[didx.at[i * NBUF + j]],
                                  sem).wait()
        return _

    lax.fori_loop(0, kcb // NBUF, body, None)
    plsc.subcore_barrier()
    sl = pl.ds(s * ROWS_PER_TILE, ROWS_PER_TILE)
    pltpu.sync_copy(acc.at[sl], out_hbm.at[c].at[sl])


def _prop_body(table_hbm, ei_hbm, zeros_hbm, out_hbm,
               sidx, didx, rows, acc, gsem, ssem):
    c = lax.axis_index("c")
    s = lax.axis_index("s")
    is_fast, base, kcb = _chunk_assignment(c, s)
    pltpu.sync_copy(zeros_hbm, acc.at[pl.ds(s * ROWS_PER_TILE, ROWS_PER_TILE)])
    _load_idx(ei_hbm, 0, is_fast, base, sidx)
    _load_idx(ei_hbm, 1, is_fast, base, didx)
    plsc.subcore_barrier()

    for j in range(NBUF):
        pltpu.async_copy(table_hbm.at[sidx.at[j]], rows.at[j], gsem.at[j])

    def body(i, _):
        for j in range(NBUF):
            k = i * NBUF + j
            pltpu.make_async_copy(table_hbm.at[sidx.at[k]], rows.at[j],
                                  gsem.at[j]).wait()
            pltpu.async_copy(rows.at[j], acc.at[didx.at[k]], ssem.at[j],
                             add=True)
            nk = k + NBUF

            @pl.when(nk < kcb)
            def _prefetch():
                pltpu.make_async_copy(rows.at[j], acc.at[didx.at[k]],
                                      ssem.at[j]).wait()
                pltpu.async_copy(table_hbm.at[sidx.at[nk]], rows.at[j],
                                 gsem.at[j])
        return _

    lax.fori_loop(0, kcb // NBUF, body, None)
    for j in range(NBUF):
        pltpu.make_async_copy(rows.at[j], acc.at[didx.at[kcb - NBUF + j]],
                              ssem.at[j]).wait()
    plsc.subcore_barrier()
    sl = pl.ds(s * ROWS_PER_TILE, ROWS_PER_TILE)
    pltpu.sync_copy(acc.at[sl], out_hbm.at[c].at[sl])


def _sc_degree(ei_r, ones128, zeros640):
    return pl.kernel(
        _deg_body,
        out_type=jax.ShapeDtypeStruct((NC, N_ACC, D), jnp.float32),
        mesh=_sc_mesh(),
        compiler_params=_SC_PARAMS,
        scratch_types=[
            pltpu.VMEM((KFB, CW), jnp.int32),
            pltpu.VMEM((CW, D), jnp.float32),
            pltpu.VMEM_SHARED((N_ACC, D), jnp.float32),
            pltpu.SemaphoreType.DMA,
        ],
    )(ei_r, ones128, zeros640)


def _sc_propagate(table, ei_r, zeros640):
    return pl.kernel(
        _prop_body,
        out_type=jax.ShapeDtypeStruct((NC, N_ACC, D), jnp.float32),
        mesh=_sc_mesh(),
        compiler_params=_SC_PARAMS,
        scratch_types=[
            pltpu.VMEM((KFB, CW), jnp.int32),
            pltpu.VMEM((KFB, CW), jnp.int32),
            pltpu.VMEM((NBUF, CW, D), jnp.float32),
            pltpu.VMEM_SHARED((N_ACC, D), jnp.float32),
            pltpu.SemaphoreType.DMA((NBUF,)),
            pltpu.SemaphoreType.DMA((NBUF,)),
        ],
    )(table, ei_r, zeros640)


def _tca_body(x_ref, w1_ref, h_ref):
    h_ref[...] = jnp.dot(x_ref[...], w1_ref[...],
                         preferred_element_type=jnp.float32)


def _tcb_body(degp_ref, h_ref, hp1_ref, dis_ref):
    deg = degp_ref[0, : N, 0:1] + degp_ref[1, : N, 0:1] + 1.0
    dis = lax.rsqrt(deg)
    hp1_ref[pl.ds(0, N), :] = dis * h_ref[...]
    hp1_ref[pl.ds(N, N_TBL - N), :] = jnp.zeros((N_TBL - N, D), jnp.float32)
    dis_ref[...] = dis


def _tc2_body(p1_ref, hp1_ref, dis_ref, b1_ref, q_ref):
    s1 = p1_ref[0, : N, :] + p1_ref[1, : N, :]
    dis = dis_ref[...]
    pre = dis * (s1 + hp1_ref[pl.ds(0, N), :]) + b1_ref[...]
    q_ref[pl.ds(0, N), :] = dis * jnp.maximum(pre, 0.0)
    q_ref[pl.ds(N, N_TBL - N), :] = jnp.zeros((N_TBL - N, D), jnp.float32)


def _tc3_body(p2_ref, q_ref, dis_ref, w2_ref, b2_ref, wc_ref, bc_ref, out_ref):
    s2 = p2_ref[0, : N, :] + p2_ref[1, : N, :]
    t = s2 + q_ref[pl.ds(0, N), :]
    conv2 = dis_ref[...] * jnp.dot(
        t, w2_ref[...], preferred_element_type=jnp.float32) + b2_ref[...]
    emb = jnp.maximum(conv2, 0.0)
    z = jnp.dot(emb, wc_ref[...], preferred_element_type=jnp.float32) + bc_ref[...]
    out_ref[...] = jax.nn.sigmoid(z)


def _tc_call(body, out_shapes, *args):
    return pl.pallas_call(body, out_shape=out_shapes)(*args)


@jax.jit
def kernel(x, edge_index, W1, b1, W2, b2, Wc, bc):
    ei_r = jnp.pad(edge_index.astype(jnp.int32), ((0, 0), (0, EP - E)),
                   constant_values=N).reshape(2, NBLK, CW)
    zeros640 = jnp.zeros((ROWS_PER_TILE, D), jnp.float32)
    ones128 = jnp.ones((CW, D), jnp.float32)

    h = _tc_call(_tca_body, jax.ShapeDtypeStruct((N, D), jnp.float32), x, W1)
    degp = _sc_degree(ei_r, ones128, zeros640)
    hp1, dis = _tc_call(
        _tcb_body,
        [jax.ShapeDtypeStruct((N_TBL, D), jnp.float32),
         jax.ShapeDtypeStruct((N, 1), jnp.float32)],
        degp, h)
    p1 = _sc_propagate(hp1, ei_r, zeros640)
    q = _tc_call(
        _tc2_body,
        jax.ShapeDtypeStruct((N_TBL, D), jnp.float32),
        p1, hp1, dis, b1.reshape(1, D))
    p2 = _sc_propagate(q, ei_r, zeros640)
    out = _tc_call(
        _tc3_body,
        jax.ShapeDtypeStruct((N, 1), jnp.float32),
        p2, q, dis, W2, b2.reshape(1, 2), Wc, bc.reshape(1, 1))
    return out
